# Initial kernel scaffold; baseline (speedup 1.0000x reference)
import jax
import jax.numpy as jnp
from jax.experimental import pallas as pl


def kernel(x, edge_index, W1, b1, W2, b2, W3, b3, Wl, bl):
    raise NotImplementedError("write your pallas kernel here")


# speedup vs baseline: 5.9802x; 5.9802x over previous
import functools

import jax
import jax.numpy as jnp
from jax import lax
from jax.experimental import pallas as pl
from jax.experimental.pallas import tpu as pltpu
from jax.experimental.pallas import tpu_sc as plsc

N = 10000
E = 320000
D = 128
H = 32
C = 3

NW = 32
CK = 128
CH = 80
EP = NW * CH * CK
NP = 10240
SL = NP // 16


@functools.lru_cache(maxsize=1)
def _sc_kernels():
  mesh = plsc.VectorSubcoreMesh(core_axis_name="c", subcore_axis_name="s")

  @functools.partial(
      pl.kernel,
      mesh=mesh,
      out_type=jax.ShapeDtypeStruct((2, NP), jnp.float32),
      scratch_types=[
          pltpu.VMEM((CH, CK), jnp.int32),
          pltpu.VMEM((CK,), jnp.float32),
          pltpu.VMEM_SHARED((NP,), jnp.float32),
      ],
  )
  def sc_deg(dst_hbm, zeros1_hbm, out_hbm, idx_v, ones_v, acc_sh):
    c = lax.axis_index("c")
    s = lax.axis_index("s")
    wid = s * 2 + c
    pltpu.sync_copy(zeros1_hbm.at[pl.ds(s * SL, SL)], acc_sh.at[pl.ds(s * SL, SL)])
    for k in range(CK // 16):
      ones_v[pl.ds(k * 16, 16)] = jnp.ones((16,), jnp.float32)
    pltpu.sync_copy(dst_hbm.at[wid], idx_v)
    plsc.subcore_barrier()

    def body(j, carry):
      pltpu.sync_copy(ones_v, acc_sh.at[idx_v.at[j]], add=True)
      return carry

    lax.fori_loop(0, CH, body, 0)
    plsc.subcore_barrier()
    pltpu.sync_copy(acc_sh.at[pl.ds(s * SL, SL)], out_hbm.at[c, pl.ds(s * SL, SL)])

  @functools.partial(
      pl.kernel,
      mesh=mesh,
      out_type=jax.ShapeDtypeStruct((2, NP), jnp.float32),
      scratch_types=[
          pltpu.VMEM((CH, CK), jnp.int32),
          pltpu.VMEM((CH, CK), jnp.int32),
          pltpu.VMEM((CK,), jnp.float32),
          pltpu.VMEM_SHARED((NP,), jnp.float32),
          pltpu.VMEM_SHARED((NP,), jnp.float32),
          pltpu.SemaphoreType.DMA,
      ],
  )
  def sc_u(dinv_hbm, src_hbm, dst_hbm, zeros1_hbm, out_hbm,
           sidx_v, didx_v, vals_v, acc_sh, dinv_sh, sem):
    c = lax.axis_index("c")
    s = lax.axis_index("s")
    wid = s * 2 + c
    pltpu.sync_copy(zeros1_hbm.at[pl.ds(s * SL, SL)], acc_sh.at[pl.ds(s * SL, SL)])
    pltpu.sync_copy(dinv_hbm.at[pl.ds(s * SL, SL)], dinv_sh.at[pl.ds(s * SL, SL)])
    pltpu.sync_copy(src_hbm.at[wid], sidx_v)
    pltpu.sync_copy(dst_hbm.at[wid], didx_v)
    plsc.subcore_barrier()

    def body(j, carry):
      pltpu.sync_copy(dinv_sh.at[didx_v.at[j]], vals_v)
      pltpu.sync_copy(vals_v, acc_sh.at[sidx_v.at[j]], add=True)
      return carry

    lax.fori_loop(0, CH, body, 0)
    plsc.subcore_barrier()
    pltpu.sync_copy(acc_sh.at[pl.ds(s * SL, SL)], out_hbm.at[c, pl.ds(s * SL, SL)])

  NB = 16
  BI = CH * H // NB
  SLF = NP * H // 16

  @functools.partial(
      pl.kernel,
      mesh=mesh,
      out_type=jax.ShapeDtypeStruct((2, NP * H), jnp.float32),
      scratch_types=[
          pltpu.VMEM((BI, CK), jnp.int32),
          pltpu.VMEM((BI, CK), jnp.int32),
          pltpu.VMEM((CK,), jnp.float32),
          pltpu.VMEM_SHARED((NP * H,), jnp.float32),
          pltpu.VMEM_SHARED((NP * H,), jnp.float32),
      ],
  )
  def sc_agg_el(yf_hbm, srcx_hbm, dstx_hbm, zerosf_hbm, out_hbm,
                sidx_v, didx_v, vals_v, acc_sh, y_sh):
    c = lax.axis_index("c")
    s = lax.axis_index("s")
    wid = s * 2 + c
    pltpu.sync_copy(zerosf_hbm.at[pl.ds(s * SLF, SLF)], acc_sh.at[pl.ds(s * SLF, SLF)])
    pltpu.sync_copy(yf_hbm.at[pl.ds(s * SLF, SLF)], y_sh.at[pl.ds(s * SLF, SLF)])
    plsc.subcore_barrier()

    def outer(ob, carry):
      pltpu.sync_copy(srcx_hbm.at[wid * NB + ob], sidx_v)
      pltpu.sync_copy(dstx_hbm.at[wid * NB + ob], didx_v)

      def body(j, cc):
        pltpu.sync_copy(y_sh.at[sidx_v.at[j]], vals_v)
        pltpu.sync_copy(vals_v, acc_sh.at[didx_v.at[j]], add=True)
        return cc

      lax.fori_loop(0, BI, body, 0)
      return carry

    lax.fori_loop(0, NB, outer, 0)
    plsc.subcore_barrier()
    pltpu.sync_copy(acc_sh.at[pl.ds(s * SLF, SLF)], out_hbm.at[c, pl.ds(s * SLF, SLF)])

  def sc_agg(yf, srcx, dstx, zerosf):
    return sc_agg_el(yf, srcx, dstx, zerosf)

  return sc_deg, sc_u, sc_agg


def _dense1_body(d0_ref, d1_ref, x_ref, w_ref, dinv_ref, y_ref):
    i = pl.program_id(0)
    deg = d0_ref[...] + d1_ref[...] + 1.0
    row = lax.broadcasted_iota(jnp.int32, (128, 1), 0) + i * 128
    dv = jnp.where(row < N, lax.rsqrt(deg), 0.0)
    dinv_ref[...] = dv
    y_ref[...] = dv * jnp.dot(x_ref[...], w_ref[...],
                              preferred_element_type=jnp.float32)


def _tc_dense1(d0, d1, xp, W1):
    return pl.pallas_call(
        _dense1_body,
        grid=(NP // 128,),
        in_specs=[
            pl.BlockSpec((128, 1), lambda i: (i, 0)),
            pl.BlockSpec((128, 1), lambda i: (i, 0)),
            pl.BlockSpec((128, D), lambda i: (i, 0)),
            pl.BlockSpec((D, H), lambda i: (0, 0)),
        ],
        out_specs=[
            pl.BlockSpec((128, 1), lambda i: (i, 0)),
            pl.BlockSpec((128, H), lambda i: (i, 0)),
        ],
        out_shape=[
            jax.ShapeDtypeStruct((NP, 1), jnp.float32),
            jax.ShapeDtypeStruct((NP, H), jnp.float32),
        ],
    )(d0, d1, xp, W1)


def _layer_body(p0_ref, p1_ref, y_ref, dinv_ref, b_ref, w_ref, ynext_ref):
    dv = dinv_ref[...]
    h = jax.nn.relu(dv * (p0_ref[...] + p1_ref[...] + y_ref[...]) + b_ref[...])
    ynext_ref[...] = dv * jnp.dot(h, w_ref[...],
                                  preferred_element_type=jnp.float32)


def _tc_layer(p0, p1, y, dinv, b, Wn):
    return pl.pallas_call(
        _layer_body,
        grid=(NP // 128,),
        in_specs=[
            pl.BlockSpec((128, H), lambda i: (i, 0)),
            pl.BlockSpec((128, H), lambda i: (i, 0)),
            pl.BlockSpec((128, H), lambda i: (i, 0)),
            pl.BlockSpec((128, 1), lambda i: (i, 0)),
            pl.BlockSpec((1, H), lambda i: (0, 0)),
            pl.BlockSpec((H, H), lambda i: (0, 0)),
        ],
        out_specs=pl.BlockSpec((128, H), lambda i: (i, 0)),
        out_shape=jax.ShapeDtypeStruct((NP, H), jnp.float32),
    )(p0, p1, y, dinv, b, Wn)


def _head_body(p0_ref, p1_ref, y_ref, dinv_ref, u0_ref, u1_ref, b2_ref,
               b3_ref, w3_ref, wl_ref, bl_ref, out_ref, acc_ref):
    i = pl.program_id(0)
    dv = dinv_ref[...]
    h2 = jax.nn.relu(dv * (p0_ref[...] + p1_ref[...] + y_ref[...]) + b2_ref[...])
    v = dv * (u0_ref[...] + u1_ref[...] + dv)
    contrib = jnp.sum(v * h2, axis=0, keepdims=True)

    @pl.when(i == 0)
    def _():
        acc_ref[...] = jnp.zeros_like(acc_ref)

    acc_ref[...] += contrib

    @pl.when(i == NP // 128 - 1)
    def _():
        g = jnp.dot(acc_ref[...], w3_ref[...],
                    preferred_element_type=jnp.float32) * (1.0 / N) + b3_ref[...]
        logits = jnp.dot(g, wl_ref[...],
                         preferred_element_type=jnp.float32) + bl_ref[...]
        m = jnp.max(logits, axis=1, keepdims=True)
        ex = jnp.exp(logits - m)
        out_ref[...] = ex / jnp.sum(ex, axis=1, keepdims=True)


def _tc_head(p0, p1, y2, dinv, u0, u1, b2, b3, W3, Wl, bl):
    return pl.pallas_call(
        _head_body,
        grid=(NP // 128,),
        in_specs=[
            pl.BlockSpec((128, H), lambda i: (i, 0)),
            pl.BlockSpec((128, H), lambda i: (i, 0)),
            pl.BlockSpec((128, H), lambda i: (i, 0)),
            pl.BlockSpec((128, 1), lambda i: (i, 0)),
            pl.BlockSpec((128, 1), lambda i: (i, 0)),
            pl.BlockSpec((128, 1), lambda i: (i, 0)),
            pl.BlockSpec((1, H), lambda i: (0, 0)),
            pl.BlockSpec((1, H), lambda i: (0, 0)),
            pl.BlockSpec((H, H), lambda i: (0, 0)),
            pl.BlockSpec((H, C), lambda i: (0, 0)),
            pl.BlockSpec((1, C), lambda i: (0, 0)),
        ],
        out_specs=pl.BlockSpec((1, C), lambda i: (0, 0)),
        out_shape=jax.ShapeDtypeStruct((1, C), jnp.float32),
        scratch_shapes=[pltpu.VMEM((1, H), jnp.float32)],
    )(p0, p1, y2, dinv, u0, u1, b2, b3, W3, Wl, bl)


def kernel(x, edge_index, W1, b1, W2, b2, W3, b3, Wl, bl):
    sc_deg, sc_u, sc_agg = _sc_kernels()

    src = edge_index[0].astype(jnp.int32)
    dst = edge_index[1].astype(jnp.int32)
    pad = N + (jnp.arange(EP - E, dtype=jnp.int32) % (NP - N))
    srcp = jnp.concatenate([src, pad])
    dstp = jnp.concatenate([dst, pad])
    src3 = srcp.reshape(NW, CH, CK)
    dst3 = dstp.reshape(NW, CH, CK)
    col = jnp.arange(H, dtype=jnp.int32)
    srcx = (srcp[:, None] * H + col).reshape(NW * 16, CH * H // 16, CK)
    dstx = (dstp[:, None] * H + col).reshape(NW * 16, CH * H // 16, CK)
    xp = jnp.concatenate([x, jnp.zeros((NP - N, D), jnp.float32)], axis=0)
    zeros1 = jnp.zeros((NP,), jnp.float32)
    zerosf = jnp.zeros((NP * H,), jnp.float32)

    degp = sc_deg(dst3, zeros1)
    d0 = degp[0].reshape(NP, 1)
    d1 = degp[1].reshape(NP, 1)
    dinv, y1 = _tc_dense1(d0, d1, xp, W1)

    up = sc_u(dinv.reshape(NP), src3, dst3, zeros1)

    p = sc_agg(y1.reshape(NP * H), srcx, dstx, zerosf)
    p = p.reshape(2, NP, H)
    y2 = _tc_layer(p[0], p[1], y1, dinv, b1.reshape(1, H), W2)

    p2 = sc_agg(y2.reshape(NP * H), srcx, dstx, zerosf)
    p2 = p2.reshape(2, NP, H)
    out = _tc_head(p2[0], p2[1], y2, dinv,
                   up[0].reshape(NP, 1), up[1].reshape(NP, 1),
                   b2.reshape(1, H), b3.reshape(1, H), W3, Wl,
                   bl.reshape(1, C))
    return out
